# baseline (speedup 1.0000x reference)
import functools

import jax
import jax.numpy as jnp
from jax import lax
from jax.experimental import pallas as pl
from jax.experimental.pallas import tpu as pltpu
from jax.experimental.pallas import tpu_sc as plsc

_NUM_CORES = 2
_NUM_SUBCORES = 16
_NUM_WORKERS = _NUM_CORES * _NUM_SUBCORES


@functools.partial(jax.jit, static_argnames=("chunk",))
def _sc_gather(table, idx, chunk=512):
    b_total = idx.shape[0]
    d = table.shape[1]
    b_per_w = b_total // _NUM_WORKERS
    n_chunks = b_per_w // chunk
    assert b_per_w * _NUM_WORKERS == b_total
    assert n_chunks * chunk == b_per_w

    mesh = plsc.VectorSubcoreMesh(core_axis_name="c", subcore_axis_name="s")

    @functools.partial(
        pl.kernel,
        mesh=mesh,
        out_type=jax.ShapeDtypeStruct((b_total, d), jnp.float32),
        scratch_types=[
            pltpu.VMEM((chunk,), jnp.int32),
            pltpu.VMEM((chunk, d), jnp.float32),
            pltpu.SemaphoreType.DMA,
        ],
        compiler_params=pltpu.CompilerParams(use_tc_tiling_on_sc=False),
    )
    def k(table_hbm, idx_hbm, out_hbm, idx_v, rows_v, sem):
        wid = lax.axis_index("s") * _NUM_CORES + lax.axis_index("c")
        base = wid * b_per_w

        def step(i, _):
            off = base + i * chunk
            pltpu.sync_copy(idx_hbm.at[pl.ds(off, chunk)], idx_v)
            pltpu.async_copy(table_hbm.at[idx_v], rows_v, sem).wait()
            pltpu.sync_copy(rows_v, out_hbm.at[pl.ds(off, chunk)])
            return 0

        lax.fori_loop(0, n_chunks, step, 0)

    return k(table, idx)


def kernel(x, table):
    b = x.size
    idx = x.reshape(b).astype(jnp.int32)
    out = _sc_gather(table, idx)
    return out.reshape(x.shape + (table.shape[1],))


# speedup vs baseline: 1.0429x; 1.0429x over previous
import functools

import jax
import jax.numpy as jnp
from jax import lax
from jax.experimental import pallas as pl
from jax.experimental.pallas import tpu as pltpu
from jax.experimental.pallas import tpu_sc as plsc

_NUM_CORES = 2
_NUM_SUBCORES = 16
_NUM_WORKERS = _NUM_CORES * _NUM_SUBCORES


@functools.partial(jax.jit, static_argnames=("chunk", "nbuf"))
def _sc_gather(table, idx, chunk=512, nbuf=2):
    b_total = idx.shape[0]
    d = table.shape[1]
    b_per_w = b_total // _NUM_WORKERS
    n_chunks = b_per_w // chunk
    n_outer = n_chunks // nbuf
    assert b_per_w * _NUM_WORKERS == b_total
    assert n_outer * nbuf == n_chunks and n_chunks * chunk == b_per_w

    mesh = plsc.VectorSubcoreMesh(core_axis_name="c", subcore_axis_name="s")

    @functools.partial(
        pl.kernel,
        mesh=mesh,
        out_type=jax.ShapeDtypeStruct((b_total, d), jnp.float32),
        scratch_types=[
            pltpu.VMEM((nbuf, chunk), jnp.int32),
            pltpu.VMEM((nbuf, chunk, d), jnp.float32),
            [pltpu.SemaphoreType.DMA] * nbuf,
            [pltpu.SemaphoreType.DMA] * nbuf,
        ],
        compiler_params=pltpu.CompilerParams(use_tc_tiling_on_sc=False),
    )
    def k(table_hbm, idx_hbm, out_hbm, idx_v, rows_v, gsems, wsems):
        wid = lax.axis_index("s") * _NUM_CORES + lax.axis_index("c")
        base = wid * b_per_w

        def fire_gather(slot, off):
            pltpu.sync_copy(idx_hbm.at[pl.ds(off, chunk)], idx_v.at[slot])
            pltpu.async_copy(
                table_hbm.at[idx_v.at[slot]], rows_v.at[slot], gsems[slot]
            )

        for b in range(nbuf):
            fire_gather(b, base + b * chunk)

        def outer(g, _):
            off0 = base + g * (nbuf * chunk)
            for b in range(nbuf):
                pltpu.make_async_copy(
                    table_hbm.at[idx_v.at[b]], rows_v.at[b], gsems[b]
                ).wait()
                pltpu.async_copy(
                    rows_v.at[b],
                    out_hbm.at[pl.ds(off0 + b * chunk, chunk)],
                    wsems[b],
                )
            for b in range(nbuf):
                pltpu.make_async_copy(
                    rows_v.at[b],
                    out_hbm.at[pl.ds(off0 + b * chunk, chunk)],
                    wsems[b],
                ).wait()

                @pl.when(g + 1 < n_outer)
                def _():
                    fire_gather(b, off0 + (nbuf + b) * chunk)

            return 0

        lax.fori_loop(0, n_outer, outer, 0)

    return k(table, idx)


def kernel(x, table):
    b = x.size
    idx = x.reshape(b).astype(jnp.int32)
    out = _sc_gather(table, idx)
    return out.reshape(x.shape + (table.shape[1],))


# speedup vs baseline: 1.0725x; 1.0284x over previous
import functools

import jax
import jax.numpy as jnp
from jax import lax
from jax.experimental import pallas as pl
from jax.experimental.pallas import tpu as pltpu
from jax.experimental.pallas import tpu_sc as plsc

_NUM_CORES = 2
_NUM_SUBCORES = 16
_NUM_WORKERS = _NUM_CORES * _NUM_SUBCORES


@functools.partial(jax.jit, static_argnames=("chunk", "nbuf"))
def _sc_gather(table, idx, chunk=512, nbuf=2):
    b_total = idx.shape[0]
    d = table.shape[1]
    b_per_w = b_total // _NUM_WORKERS
    n_chunks = b_per_w // chunk
    n_outer = n_chunks // nbuf
    assert b_per_w * _NUM_WORKERS == b_total
    assert n_outer * nbuf == n_chunks and n_chunks * chunk == b_per_w

    mesh = plsc.VectorSubcoreMesh(core_axis_name="c", subcore_axis_name="s")

    @functools.partial(
        pl.kernel,
        mesh=mesh,
        out_type=jax.ShapeDtypeStruct((b_total, d), jnp.float32),
        scratch_types=[
            pltpu.VMEM((nbuf, chunk), jnp.int32),
            pltpu.VMEM((nbuf, chunk, d), jnp.float32),
            [pltpu.SemaphoreType.DMA] * nbuf,
            [pltpu.SemaphoreType.DMA] * nbuf,
        ],
        compiler_params=pltpu.CompilerParams(use_tc_tiling_on_sc=False),
    )
    def k(table_hbm, idx_hbm, out_hbm, idx_v, rows_v, gsems, wsems):
        wid = lax.axis_index("s") * _NUM_CORES + lax.axis_index("c")
        base = wid * b_per_w

        def fire_gather(slot, off):
            pltpu.sync_copy(idx_hbm.at[pl.ds(off, chunk)], idx_v.at[slot])
            pltpu.async_copy(
                table_hbm.at[idx_v.at[slot]], rows_v.at[slot], gsems[slot]
            )

        for b in range(nbuf):
            fire_gather(b, base + b * chunk)

        def outer(g, _):
            off0 = base + g * (nbuf * chunk)
            for b in range(nbuf):
                pltpu.make_async_copy(
                    table_hbm.at[idx_v.at[b]], rows_v.at[b], gsems[b]
                ).wait()
                pltpu.async_copy(
                    rows_v.at[b],
                    out_hbm.at[pl.ds(off0 + b * chunk, chunk)],
                    wsems[b],
                )
            for b in range(nbuf):
                pltpu.make_async_copy(
                    rows_v.at[b],
                    out_hbm.at[pl.ds(off0 + b * chunk, chunk)],
                    wsems[b],
                ).wait()

                @pl.when(g + 1 < n_outer)
                def _():
                    fire_gather(b, off0 + (nbuf + b) * chunk)

            return 0

        lax.fori_loop(0, n_outer, outer, 0)

    return k(table, idx)


def kernel(x, table):
    n_tok, n_seq = x.shape
    d = table.shape[1]
    idx = x.T.reshape(n_tok * n_seq).astype(jnp.int32)
    out = _sc_gather(table, idx)
    return out.reshape(n_seq, n_tok, d).transpose(1, 0, 2)
